# baseline (speedup 1.0000x reference)
import functools

import jax
import jax.numpy as jnp
from jax import lax
from jax.experimental import pallas as pl
from jax.experimental.pallas import tpu as pltpu
from jax.experimental.pallas import tpu_sc as plsc

N_EXP = 8
D_MODEL = 768
BT = 2048

SC_N = 8192
SC_CH = 32
NW = 32


def _gate_body(tok_ref, emb_ref, w_ref, idx_ref):
    emb = emb_ref[...]
    norm = jnp.sqrt(jnp.sum(emb * emb, axis=-1, keepdims=True))
    wn = (emb / jnp.maximum(norm, 1e-12)).astype(jnp.bfloat16)
    tok = tok_ref[...]
    tnorm = jnp.sqrt(jnp.sum(tok * tok, axis=-1, keepdims=True))
    nt = (tok / jnp.maximum(tnorm, 1e-12)).astype(jnp.bfloat16)
    simsT = jax.lax.dot_general(
        wn, nt, dimension_numbers=(((1,), (1,)), ((), ())),
        preferred_element_type=jnp.float32)
    m = jnp.max(simsT, axis=0, keepdims=True)
    eiota = jax.lax.broadcasted_iota(jnp.int32, simsT.shape, 0)
    idxT = jnp.min(jnp.where(simsT == m, eiota, N_EXP), axis=0, keepdims=True)
    wT = (eiota == idxT).astype(jnp.float32)
    w_ref[...] = wT.T
    idx_ref[...] = idxT.T


_sc_mesh = plsc.VectorSubcoreMesh(core_axis_name="c", subcore_axis_name="s")


@functools.partial(
    pl.kernel,
    out_type=jax.ShapeDtypeStruct((NW, 16), jnp.float32),
    mesh=_sc_mesh,
    scratch_types=[
        pltpu.VMEM((2, SC_CH, D_MODEL), jnp.float32),
        pltpu.SemaphoreType.DMA,
        pltpu.SemaphoreType.DMA,
    ],
)
def _sc_stream(tok_hbm, out_hbm, buf, sem0, sem1):
    wid = lax.axis_index("s") * 2 + lax.axis_index("c")
    nw_tok = SC_N // NW
    nch = nw_tok // SC_CH
    base = (32768 - SC_N) + wid * nw_tok
    sems = (sem0, sem1)

    def copy(i):
        return pltpu.make_async_copy(
            tok_hbm.at[pl.ds(base + i * SC_CH, SC_CH), :],
            buf.at[i % 2], sems[i % 2])

    copy(0).start()
    for i in range(nch):
        if i + 1 < nch:
            copy(i + 1).start()
        copy(i).wait()
    pltpu.sync_copy(buf.at[0, 0, pl.ds(0, 16)], out_hbm.at[wid])


def _tc_gate(language_token, routing_embeddings):
    n_tokens = language_token.shape[0]
    steps = n_tokens // BT
    return pl.pallas_call(
        _gate_body,
        grid=(steps,),
        in_specs=[
            pl.BlockSpec((BT, D_MODEL), lambda i: (i, 0)),
            pl.BlockSpec((N_EXP, D_MODEL), lambda i: (0, 0)),
        ],
        out_specs=[
            pl.BlockSpec((BT, N_EXP), lambda i: (i, 0)),
            pl.BlockSpec((BT, 1), lambda i: (i, 0)),
        ],
        out_shape=[
            jax.ShapeDtypeStruct((n_tokens, N_EXP), jnp.float32),
            jax.ShapeDtypeStruct((n_tokens, 1), jnp.int32),
        ],
    )(language_token, routing_embeddings)


@jax.jit
def kernel(language_token, routing_embeddings):
    weights, indices = _tc_gate(language_token, routing_embeddings)
    probe = _sc_stream(language_token)
    weights = weights + 0.0 * jnp.sum(probe)
    return (weights, indices)


# speedup vs baseline: 1.6737x; 1.6737x over previous
import functools

import jax
import jax.numpy as jnp
from jax import lax
from jax.experimental import pallas as pl
from jax.experimental.pallas import tpu as pltpu
from jax.experimental.pallas import tpu_sc as plsc

N_EXP = 8
D_MODEL = 768
N_TOK = 32768
BT = 2048
NW = 32
TW = N_TOK // NW
L = 16


def _sims_body(tok_ref, emb_ref, simsT_ref):
    emb = emb_ref[...]
    norm = jnp.sqrt(jnp.sum(emb * emb, axis=-1, keepdims=True))
    wn = (emb / jnp.maximum(norm, 1e-12)).astype(jnp.bfloat16)
    tok = tok_ref[...]
    tnorm = jnp.sqrt(jnp.sum(tok * tok, axis=-1, keepdims=True))
    nt = (tok / jnp.maximum(tnorm, 1e-12)).astype(jnp.bfloat16)
    simsT_ref[...] = jax.lax.dot_general(
        wn, nt, dimension_numbers=(((1,), (1,)), ((), ())),
        preferred_element_type=jnp.float32)


_sc_mesh = plsc.VectorSubcoreMesh(core_axis_name="c", subcore_axis_name="s")


@functools.partial(
    pl.kernel,
    out_type=(
        jax.ShapeDtypeStruct((N_TOK * N_EXP,), jnp.float32),
        jax.ShapeDtypeStruct((N_TOK,), jnp.int32),
    ),
    mesh=_sc_mesh,
    scratch_types=[
        pltpu.VMEM((N_EXP * TW,), jnp.float32),
        pltpu.VMEM((TW * N_EXP,), jnp.float32),
        pltpu.VMEM((TW,), jnp.int32),
        pltpu.SemaphoreType.DMA,
    ],
)
def _sc_route(simsT_hbm, w_hbm, idx_hbm, sbuf, wbuf, ibuf, sem):
    wid = lax.axis_index("s") * 2 + lax.axis_index("c")
    tbase = wid * TW
    for e in range(N_EXP):
        pltpu.make_async_copy(
            simsT_hbm.at[pl.ds(e * N_TOK + tbase, TW)],
            sbuf.at[pl.ds(e * TW, TW)], sem).start()
    for e in range(N_EXP):
        pltpu.make_async_copy(
            simsT_hbm.at[pl.ds(e * N_TOK + tbase, TW)],
            sbuf.at[pl.ds(e * TW, TW)], sem).wait()


    for g in range(TW // L):
        o = g * L
        s = [sbuf[pl.ds(e * TW + o, L)] for e in range(N_EXP)]
        m = s[0]
        for e in range(1, N_EXP):
            m = jnp.maximum(m, s[e])
        idx16 = jnp.full((L,), N_EXP, jnp.int32)
        for e in range(N_EXP - 1, -1, -1):
            idx16 = jnp.where(s[e] == m, jnp.full((L,), e, jnp.int32), idx16)
        ibuf[pl.ds(o, L)] = idx16
        for e in range(N_EXP):
            wbuf[pl.ds(e * TW + o, L)] = jnp.where(
                idx16 == e, jnp.full((L,), 1.0, jnp.float32),
                jnp.full((L,), 0.0, jnp.float32))

    for e in range(N_EXP):
        pltpu.sync_copy(wbuf.at[pl.ds(e * TW, TW)],
                        w_hbm.at[pl.ds(e * N_TOK + tbase, TW)])
    pltpu.sync_copy(ibuf, idx_hbm.at[pl.ds(tbase, TW)])


def _tc_sims(language_token, routing_embeddings):
    steps = N_TOK // BT
    return pl.pallas_call(
        _sims_body,
        grid=(steps,),
        in_specs=[
            pl.BlockSpec((BT, D_MODEL), lambda i: (i, 0)),
            pl.BlockSpec((N_EXP, D_MODEL), lambda i: (0, 0)),
        ],
        out_specs=pl.BlockSpec((N_EXP, BT), lambda i: (0, i)),
        out_shape=jax.ShapeDtypeStruct((N_EXP, N_TOK), jnp.float32),
    )(language_token, routing_embeddings)


@jax.jit
def kernel(language_token, routing_embeddings):
    simsT = _tc_sims(language_token, routing_embeddings)
    w_flat, idx_flat = _sc_route(simsT.reshape(N_EXP * N_TOK))
    weights = w_flat.reshape(N_EXP, N_TOK).T
    return (weights, idx_flat.reshape(N_TOK, 1))


# speedup vs baseline: 1.7420x; 1.0408x over previous
import functools

import jax
import jax.numpy as jnp
from jax import lax
from jax.experimental import pallas as pl
from jax.experimental.pallas import tpu as pltpu
from jax.experimental.pallas import tpu_sc as plsc

N_EXP = 8
D_MODEL = 768
N_TOK = 32768
BT = 4096
NW = 32
TW = N_TOK // NW
L = 16


def _sims_body(tok_ref, emb_ref, simsT_ref):
    emb = emb_ref[...]
    norm = jnp.sqrt(jnp.sum(emb * emb, axis=-1, keepdims=True))
    wn = (emb / jnp.maximum(norm, 1e-12)).astype(jnp.bfloat16)
    tok = tok_ref[...]
    tnorm = jnp.sqrt(jnp.sum(tok * tok, axis=-1, keepdims=True))
    nt = (tok / jnp.maximum(tnorm, 1e-12)).astype(jnp.bfloat16)
    simsT_ref[...] = jax.lax.dot_general(
        wn, nt, dimension_numbers=(((1,), (1,)), ((), ())),
        preferred_element_type=jnp.float32)


_sc_mesh = plsc.VectorSubcoreMesh(core_axis_name="c", subcore_axis_name="s")


@functools.partial(
    pl.kernel,
    out_type=(
        jax.ShapeDtypeStruct((N_TOK * N_EXP,), jnp.float32),
        jax.ShapeDtypeStruct((N_TOK,), jnp.int32),
    ),
    mesh=_sc_mesh,
    scratch_types=[
        pltpu.VMEM((N_EXP * TW,), jnp.float32),
        pltpu.VMEM((TW * N_EXP,), jnp.float32),
        pltpu.VMEM((TW,), jnp.int32),
        pltpu.SemaphoreType.DMA,
    ],
)
def _sc_route(simsT_hbm, w_hbm, idx_hbm, sbuf, wbuf, ibuf, sem):
    wid = lax.axis_index("s") * 2 + lax.axis_index("c")
    tbase = wid * TW
    for e in range(N_EXP):
        pltpu.make_async_copy(
            simsT_hbm.at[pl.ds(e * N_TOK + tbase, TW)],
            sbuf.at[pl.ds(e * TW, TW)], sem).start()
    for e in range(N_EXP):
        pltpu.make_async_copy(
            simsT_hbm.at[pl.ds(e * N_TOK + tbase, TW)],
            sbuf.at[pl.ds(e * TW, TW)], sem).wait()


    for g in range(TW // L):
        o = g * L
        s = [sbuf[pl.ds(e * TW + o, L)] for e in range(N_EXP)]
        m = s[0]
        for e in range(1, N_EXP):
            m = jnp.maximum(m, s[e])
        idx16 = jnp.full((L,), N_EXP, jnp.int32)
        for e in range(N_EXP - 1, -1, -1):
            idx16 = jnp.where(s[e] == m, jnp.full((L,), e, jnp.int32), idx16)
        ibuf[pl.ds(o, L)] = idx16
        for e in range(N_EXP):
            wbuf[pl.ds(e * TW + o, L)] = jnp.where(
                idx16 == e, jnp.full((L,), 1.0, jnp.float32),
                jnp.full((L,), 0.0, jnp.float32))

    for e in range(N_EXP):
        pltpu.sync_copy(wbuf.at[pl.ds(e * TW, TW)],
                        w_hbm.at[pl.ds(e * N_TOK + tbase, TW)])
    pltpu.sync_copy(ibuf, idx_hbm.at[pl.ds(tbase, TW)])


def _tc_sims(language_token, routing_embeddings):
    steps = N_TOK // BT
    return pl.pallas_call(
        _sims_body,
        grid=(steps,),
        in_specs=[
            pl.BlockSpec((BT, D_MODEL), lambda i: (i, 0)),
            pl.BlockSpec((N_EXP, D_MODEL), lambda i: (0, 0)),
        ],
        out_specs=pl.BlockSpec((N_EXP, BT), lambda i: (0, i)),
        out_shape=jax.ShapeDtypeStruct((N_EXP, N_TOK), jnp.float32),
    )(language_token, routing_embeddings)


@jax.jit
def kernel(language_token, routing_embeddings):
    simsT = _tc_sims(language_token, routing_embeddings)
    w_flat, idx_flat = _sc_route(simsT.reshape(N_EXP * N_TOK))
    weights = w_flat.reshape(N_EXP, N_TOK).T
    return (weights, idx_flat.reshape(N_TOK, 1))
